# baseline (speedup 1.0000x reference)
import functools

import jax
import jax.numpy as jnp
from jax import lax
from jax.experimental import pallas as pl
from jax.experimental.pallas import tpu as pltpu
from jax.experimental.pallas import tpu_sc as plsc

VOCAB = 1000000
D = 64
B = 1024
S = 200
T = 50

NC = 2
NS = 16
NW = NC * NS

CHUNK = 128
N_ROWS = B * S + B * T
CPW = -(-N_ROWS // (NW * CHUNK))
ROWS_PW = CPW * CHUNK
N_PAD = NW * ROWS_PW


def _gather_sc(idx, table):
    mesh = plsc.VectorSubcoreMesh(core_axis_name="c", subcore_axis_name="s")

    @functools.partial(
        pl.kernel,
        mesh=mesh,
        out_type=jax.ShapeDtypeStruct((NW * CPW, CHUNK, D), jnp.float32),
        scratch_types=[
            pltpu.VMEM((CPW, CHUNK), jnp.int32),
            pltpu.VMEM((CHUNK, D), jnp.float32),
            pltpu.SemaphoreType.DMA,
        ],
        compiler_params=pltpu.CompilerParams(use_tc_tiling_on_sc=False),
    )
    def k(idx_hbm, table_hbm, out_hbm, idx_v, rows_v, sem):
        wid = lax.axis_index("s") * NC + lax.axis_index("c")
        pltpu.sync_copy(idx_hbm.at[wid], idx_v)

        def body(j, carry):
            pltpu.async_copy(table_hbm.at[idx_v.at[j]], rows_v, sem).wait()
            pltpu.sync_copy(rows_v, out_hbm.at[wid * CPW + j])
            return carry

        lax.fori_loop(0, CPW, body, 0)

    return k(idx, table)


def _dense_block(src_ref, tgt_ref, wenc_ref, v_ref, wdec_ref, out_ref, *, bb):
    src = src_ref[...]
    h = jnp.tanh(jnp.dot(src, wenc_ref[...],
                         preferred_element_type=jnp.float32))
    logits = jnp.dot(h, v_ref[...],
                     preferred_element_type=jnp.float32)
    logits = logits.reshape(bb, S)
    m = jnp.max(logits, axis=1, keepdims=True)
    e = jnp.exp(logits - m)
    attn = e / jnp.sum(e, axis=1, keepdims=True)
    hb = h.reshape(bb, S, D)
    context = jnp.sum(attn[:, :, None] * hb, axis=1)

    tgt = tgt_ref[...]
    th = jnp.tanh(jnp.dot(tgt, wdec_ref[...],
                          preferred_element_type=jnp.float32))
    thb = th.reshape(bb, T, D)
    out_ref[...] = jnp.sum(thb * context[:, None, :], axis=2)


def _dense_tc(emb, W_enc, v_attn, W_dec):
    bb = 64
    grid = B // bb
    tgt_block0 = (B * S) // (bb * T)
    return pl.pallas_call(
        functools.partial(_dense_block, bb=bb),
        grid=(grid,),
        in_specs=[
            pl.BlockSpec((bb * S, D), lambda i: (i, 0)),
            pl.BlockSpec((bb * T, D), lambda i: (tgt_block0 + i, 0)),
            pl.BlockSpec((D, D), lambda i: (0, 0)),
            pl.BlockSpec((D, 1), lambda i: (0, 0)),
            pl.BlockSpec((D, D), lambda i: (0, 0)),
        ],
        out_specs=pl.BlockSpec((bb, T), lambda i: (i, 0)),
        out_shape=jax.ShapeDtypeStruct((B, T), jnp.float32),
    )(emb, emb, W_enc, v_attn, W_dec)


def kernel(source, target, word_lut, W_enc, v_attn, W_dec):
    idx = jnp.concatenate([source.reshape(-1), target.reshape(-1)])
    idx = jnp.pad(idx, (0, N_PAD - N_ROWS)).reshape(NW, CPW, CHUNK)
    emb = _gather_sc(idx, word_lut).reshape(N_PAD, D)
    return _dense_tc(emb, W_enc, v_attn.reshape(D, 1), W_dec)


# speedup vs baseline: 1.0776x; 1.0776x over previous
import functools

import jax
import jax.numpy as jnp
from jax import lax
from jax.experimental import pallas as pl
from jax.experimental.pallas import tpu as pltpu
from jax.experimental.pallas import tpu_sc as plsc

VOCAB = 1000000
D = 64
B = 1024
S = 200
T = 50

NC = 2
NS = 16
NW = NC * NS

CHUNK = 128
N_ROWS = B * S + B * T
CPW = -(-N_ROWS // (NW * CHUNK))
ROWS_PW = CPW * CHUNK
N_PAD = NW * ROWS_PW
G = 3
NGRP = CPW // G


def _gather_sc(idx, table):
    mesh = plsc.VectorSubcoreMesh(core_axis_name="c", subcore_axis_name="s")

    @functools.partial(
        pl.kernel,
        mesh=mesh,
        out_type=jax.ShapeDtypeStruct((NW * CPW, CHUNK, D), jnp.float32),
        scratch_types=[
            pltpu.VMEM((CPW, CHUNK), jnp.int32),
            pltpu.VMEM((2 * G, CHUNK, D), jnp.float32),
            pltpu.SemaphoreType.DMA,
            pltpu.SemaphoreType.DMA,
        ],
        compiler_params=pltpu.CompilerParams(use_tc_tiling_on_sc=False),
    )
    def k(idx_hbm, table_hbm, out_hbm, idx_v, bufs, sem_g, sem_s):
        wid = lax.axis_index("s") * NC + lax.axis_index("c")
        pltpu.sync_copy(idx_hbm.at[wid], idx_v)
        base = wid * CPW

        def gather_grp(g):
            half = lax.rem(g, 2) * G
            for i in range(G):
                pltpu.async_copy(
                    table_hbm.at[idx_v.at[g * G + i]], bufs.at[half + i],
                    sem_g)

        def gwait_grp():
            for _ in range(G):
                pltpu.make_async_copy(
                    table_hbm.at[pl.ds(0, CHUNK)], bufs.at[0], sem_g).wait()

        def store_grp(g):
            half = lax.rem(g, 2) * G
            for i in range(G):
                pltpu.async_copy(
                    bufs.at[half + i], out_hbm.at[base + g * G + i], sem_s)

        def swait_grp():
            for _ in range(G):
                pltpu.make_async_copy(
                    table_hbm.at[pl.ds(0, CHUNK)], bufs.at[0], sem_s).wait()

        gather_grp(0)
        gwait_grp()
        store_grp(0)
        gather_grp(1)

        def body(g, carry):
            gwait_grp()
            swait_grp()
            store_grp(g)
            gather_grp(g + 1)
            return carry

        lax.fori_loop(1, NGRP - 1, body, 0)

        gwait_grp()
        swait_grp()
        store_grp(NGRP - 1)
        swait_grp()

    return k(idx, table)


def _dense_block(src_ref, tgt_ref, wenc_ref, v_ref, wdec_ref, out_ref, *, bb):
    src = src_ref[...]
    h = jnp.tanh(jnp.dot(src, wenc_ref[...],
                         preferred_element_type=jnp.float32))
    logits = jnp.dot(h, v_ref[...],
                     preferred_element_type=jnp.float32)
    l3 = logits.reshape(bb, S, 1)
    m = jnp.max(l3, axis=1, keepdims=True)
    e = jnp.exp(l3 - m)
    ssum = jnp.sum(e, axis=1, keepdims=True)
    attn = (e / ssum).reshape(bb * S, 1)
    lw = h * attn
    context = jnp.sum(lw.reshape(bb, S, D), axis=1)

    tgt = tgt_ref[...]
    th = jnp.tanh(jnp.dot(tgt, wdec_ref[...],
                          preferred_element_type=jnp.float32))
    crep = jnp.broadcast_to(context.reshape(bb, 1, D), (bb, T, D))
    out_ref[...] = jnp.sum(th.reshape(bb, T, D) * crep, axis=2)


def _dense_tc(emb, W_enc, v_attn, W_dec):
    bb = 64
    grid = B // bb
    tgt_block0 = (B * S) // (bb * T)
    return pl.pallas_call(
        functools.partial(_dense_block, bb=bb),
        grid=(grid,),
        in_specs=[
            pl.BlockSpec((bb * S, D), lambda i: (i, 0)),
            pl.BlockSpec((bb * T, D), lambda i: (tgt_block0 + i, 0)),
            pl.BlockSpec((D, D), lambda i: (0, 0)),
            pl.BlockSpec((D, 1), lambda i: (0, 0)),
            pl.BlockSpec((D, D), lambda i: (0, 0)),
        ],
        out_specs=pl.BlockSpec((bb, T), lambda i: (i, 0)),
        out_shape=jax.ShapeDtypeStruct((B, T), jnp.float32),
    )(emb, emb, W_enc, v_attn, W_dec)


def kernel(source, target, word_lut, W_enc, v_attn, W_dec):
    wl2 = lax.optimization_barrier(word_lut.reshape(VOCAB // 2, 2 * D))
    wl_lin = wl2.reshape(VOCAB, D)
    idx = jnp.concatenate([source.reshape(-1), target.reshape(-1)])
    idx = jnp.pad(idx, (0, N_PAD - N_ROWS)).reshape(NW, CPW, CHUNK)
    emb = _gather_sc(idx, wl_lin).reshape(N_PAD, D)
    return _dense_tc(emb, W_enc, v_attn.reshape(D, 1), W_dec)


# speedup vs baseline: 2.1677x; 2.0117x over previous
import functools

import jax
import jax.numpy as jnp
from jax import lax
from jax.experimental import pallas as pl
from jax.experimental.pallas import tpu as pltpu
from jax.experimental.pallas import tpu_sc as plsc

VOCAB = 1000000
D = 64
B = 1024
S = 200
T = 50

NC = 2
NS = 16
NW = NC * NS

CHUNK = 128
N_ROWS = B * S + B * T
CPW = -(-N_ROWS // (NW * CHUNK))
ROWS_PW = CPW * CHUNK
N_PAD = NW * ROWS_PW
G = 3
NGRP = CPW // G
NBUFG = 4

VB = 16384
HB = VB // 2
NVB = -(-VOCAB // VB)
V_LIN = NVB * VB


def _transpose_block(in_ref, out_ref):
    x = in_ref[...]
    xt = x.T
    out_ref[...] = jnp.concatenate([xt[:HB], xt[HB:]], axis=1)


def _transpose_tc(wl_t):
    return pl.pallas_call(
        _transpose_block,
        grid=(NVB,),
        in_specs=[pl.BlockSpec((D, VB), lambda i: (0, i))],
        out_specs=pl.BlockSpec((HB, 2 * D), lambda i: (i, 0)),
        out_shape=jax.ShapeDtypeStruct((V_LIN // 2, 2 * D), jnp.float32),
    )(wl_t)


def _gather_sc(idx, table):
    mesh = plsc.VectorSubcoreMesh(core_axis_name="c", subcore_axis_name="s")

    @functools.partial(
        pl.kernel,
        mesh=mesh,
        out_type=jax.ShapeDtypeStruct((NW * CPW, CHUNK, D), jnp.float32),
        scratch_types=[
            pltpu.VMEM((CPW, CHUNK), jnp.int32),
            pltpu.VMEM((NBUFG * G, CHUNK, D), jnp.float32),
            pltpu.SemaphoreType.DMA((NBUFG,)),
            pltpu.SemaphoreType.DMA((NBUFG,)),
        ],
        compiler_params=pltpu.CompilerParams(use_tc_tiling_on_sc=False),
    )
    def k(idx_hbm, table_hbm, out_hbm, idx_v, bufs, sem_g, sem_s):
        wid = lax.axis_index("s") * NC + lax.axis_index("c")
        pltpu.sync_copy(idx_hbm.at[wid], idx_v)
        base = wid * CPW

        def gather_grp(g):
            k = lax.rem(g, NBUFG)
            for i in range(G):
                pltpu.async_copy(
                    table_hbm.at[idx_v.at[g * G + i]], bufs.at[k * G + i],
                    sem_g.at[k])

        def gwait_grp(g):
            k = lax.rem(g, NBUFG)
            for _ in range(G):
                pltpu.make_async_copy(
                    table_hbm.at[pl.ds(0, CHUNK)], bufs.at[0],
                    sem_g.at[k]).wait()

        def store_grp(g):
            k = lax.rem(g, NBUFG)
            for i in range(G):
                pltpu.async_copy(
                    bufs.at[k * G + i], out_hbm.at[base + g * G + i],
                    sem_s.at[k])

        def swait_grp(g):
            k = lax.rem(g, NBUFG)
            for _ in range(G):
                pltpu.make_async_copy(
                    table_hbm.at[pl.ds(0, CHUNK)], bufs.at[0],
                    sem_s.at[k]).wait()

        gather_grp(0)
        gather_grp(1)
        gather_grp(2)
        gwait_grp(0)
        store_grp(0)
        gather_grp(3)

        def body(g, carry):
            gwait_grp(g)
            swait_grp(g - 1)
            store_grp(g)
            gather_grp(g + 3)
            return carry

        lax.fori_loop(1, NGRP - 3, body, 0)

        for g in (NGRP - 3, NGRP - 2, NGRP - 1):
            gwait_grp(g)
            swait_grp(g - 1)
            store_grp(g)
        swait_grp(NGRP - 1)

    return k(idx, table)


def _dense_block(src_ref, tgt_ref, w2e_ref, v2_ref, w2d_ref, k2_ref,
                 outa_ref, outb_ref, *, bb):
    sh = bb * S // 2
    x = src_ref[...]
    h2 = jnp.tanh(jnp.dot(x, w2e_ref[...],
                          preferred_element_type=jnp.float32))
    lg = jnp.dot(h2, v2_ref[...],
                 preferred_element_type=jnp.float32)
    l3 = lg.reshape(bb, S // 2, 2)
    m = jnp.max(jnp.max(l3, axis=1, keepdims=True),
                axis=2, keepdims=True)
    e = jnp.exp(l3 - m)
    ssum = jnp.sum(jnp.sum(e, axis=1, keepdims=True),
                   axis=2, keepdims=True)
    attn2 = (e / ssum).reshape(sh, 2)
    arep = jnp.dot(attn2, k2_ref[...],
                   preferred_element_type=jnp.float32)
    c128 = jnp.sum((h2 * arep).reshape(bb, S // 2, 2 * D), axis=1)
    ctx = c128[:, :D] + c128[:, D:]

    t2 = tgt_ref[...]
    th2 = jnp.tanh(jnp.dot(t2, w2d_ref[...],
                           preferred_element_type=jnp.float32))
    cc = jnp.concatenate([ctx, ctx], axis=1)
    prod = (th2.reshape(bb, T // 2, 2 * D)
            * jnp.broadcast_to(cc.reshape(bb, 1, 2 * D),
                               (bb, T // 2, 2 * D)))
    outa_ref[...] = jnp.sum(prod[:, :, :D], axis=2)
    outb_ref[...] = jnp.sum(prod[:, :, D:], axis=2)


def _dense_tc(emb2, W_enc, v_attn, W_dec):
    bb = 64
    grid = B // bb
    sh = bb * S // 2
    th = bb * T // 2
    tgt_block0 = (B * S // 2) // th

    z = jnp.zeros((D, D), jnp.float32)
    w2e = jnp.block([[W_enc, z], [z, W_enc]])
    w2d = jnp.block([[W_dec, z], [z, W_dec]])
    zv = jnp.zeros((D, 1), jnp.float32)
    v2 = jnp.block([[v_attn, zv], [zv, v_attn]])
    o = jnp.ones((1, D), jnp.float32)
    zo = jnp.zeros((1, D), jnp.float32)
    k2 = jnp.block([[o, zo], [zo, o]])

    outa, outb = pl.pallas_call(
        functools.partial(_dense_block, bb=bb),
        grid=(grid,),
        in_specs=[
            pl.BlockSpec((sh, 2 * D), lambda i: (i, 0)),
            pl.BlockSpec((th, 2 * D), lambda i: (tgt_block0 + i, 0)),
            pl.BlockSpec((2 * D, 2 * D), lambda i: (0, 0)),
            pl.BlockSpec((2 * D, 2), lambda i: (0, 0)),
            pl.BlockSpec((2 * D, 2 * D), lambda i: (0, 0)),
            pl.BlockSpec((2, 2 * D), lambda i: (0, 0)),
        ],
        out_specs=[
            pl.BlockSpec((bb, T // 2), lambda i: (i, 0)),
            pl.BlockSpec((bb, T // 2), lambda i: (i, 0)),
        ],
        out_shape=[
            jax.ShapeDtypeStruct((B, T // 2), jnp.float32),
            jax.ShapeDtypeStruct((B, T // 2), jnp.float32),
        ],
    )(emb2, emb2, w2e, v2, w2d, k2)
    return jnp.stack([outa, outb], axis=2).reshape(B, T)


def kernel(source, target, word_lut, W_enc, v_attn, W_dec):
    wl2 = _transpose_tc(word_lut.T)
    wl_lin = wl2.reshape(V_LIN, D)
    idx = jnp.concatenate([source.reshape(-1), target.reshape(-1)])
    tb = idx & (VB - 1)
    idx = (idx & ~(VB - 1)) | (((tb << 1) & (VB - 1)) | (tb >> 13))
    idx = jnp.pad(idx, (0, N_PAD - N_ROWS)).reshape(NW, CPW, CHUNK)
    emb = _gather_sc(idx, wl_lin)
    emb2 = emb.reshape(N_PAD // 2, 2 * D)
    return _dense_tc(emb2, W_enc, v_attn.reshape(D, 1), W_dec)


# speedup vs baseline: 2.4503x; 1.1303x over previous
import functools

import jax
import jax.numpy as jnp
from jax import lax
from jax.experimental import pallas as pl
from jax.experimental.pallas import tpu as pltpu
from jax.experimental.pallas import tpu_sc as plsc

VOCAB = 1000000
D = 64
B = 1024
S = 200
T = 50

NC = 2
NS = 16
NW = NC * NS

CHUNK = 128
N_ROWS = B * S + B * T
CPW = -(-N_ROWS // (NW * CHUNK))
ROWS_PW = CPW * CHUNK
N_PAD = NW * ROWS_PW
G = 3
NGRP = CPW // G
NBUFG = 4

VB = 16384
HB = VB // 2
NVB = -(-VOCAB // VB)
V_LIN = NVB * VB


def _transpose_block(in_ref, out_ref):
    x = in_ref[...]
    xt = x.T
    out_ref[...] = jnp.concatenate([xt[:HB], xt[HB:]], axis=1)


def _transpose_tc(wl_t):
    return pl.pallas_call(
        _transpose_block,
        grid=(NVB,),
        in_specs=[pl.BlockSpec((D, VB), lambda i: (0, i))],
        out_specs=pl.BlockSpec((HB, 2 * D), lambda i: (i, 0)),
        out_shape=jax.ShapeDtypeStruct((V_LIN // 2, 2 * D), jnp.float32),
    )(wl_t)


def _gather_sc(idx, table):
    mesh = plsc.VectorSubcoreMesh(core_axis_name="c", subcore_axis_name="s")

    @functools.partial(
        pl.kernel,
        mesh=mesh,
        out_type=jax.ShapeDtypeStruct((NW * CPW, CHUNK, D), jnp.float32),
        scratch_types=[
            pltpu.VMEM((CPW, CHUNK), jnp.int32),
            pltpu.VMEM((NBUFG * G, CHUNK, D), jnp.float32),
            pltpu.SemaphoreType.DMA((NBUFG,)),
            pltpu.SemaphoreType.DMA((NBUFG,)),
        ],
        compiler_params=pltpu.CompilerParams(use_tc_tiling_on_sc=False),
    )
    def k(idx_hbm, table_hbm, out_hbm, idx_v, bufs, sem_g, sem_s):
        wid = lax.axis_index("s") * NC + lax.axis_index("c")
        pltpu.sync_copy(idx_hbm.at[wid], idx_v)
        base = wid * CPW

        def gather_grp(g):
            k = lax.rem(g, NBUFG)
            for i in range(G):
                pltpu.async_copy(
                    table_hbm.at[idx_v.at[g * G + i]], bufs.at[k * G + i],
                    sem_g.at[k])

        def gwait_grp(g):
            k = lax.rem(g, NBUFG)
            for _ in range(G):
                pltpu.make_async_copy(
                    table_hbm.at[pl.ds(0, CHUNK)], bufs.at[0],
                    sem_g.at[k]).wait()

        def store_grp(g):
            k = lax.rem(g, NBUFG)
            for i in range(G):
                pltpu.async_copy(
                    bufs.at[k * G + i], out_hbm.at[base + g * G + i],
                    sem_s.at[k])

        def swait_grp(g):
            k = lax.rem(g, NBUFG)
            for _ in range(G):
                pltpu.make_async_copy(
                    table_hbm.at[pl.ds(0, CHUNK)], bufs.at[0],
                    sem_s.at[k]).wait()

        gather_grp(0)
        gather_grp(1)
        gather_grp(2)
        gwait_grp(0)
        store_grp(0)
        gather_grp(3)

        def body(g, carry):
            gwait_grp(g)
            swait_grp(g - 1)
            store_grp(g)
            gather_grp(g + 3)
            return carry

        lax.fori_loop(1, NGRP - 3, body, 0)

        for g in (NGRP - 3, NGRP - 2, NGRP - 1):
            gwait_grp(g)
            swait_grp(g - 1)
            store_grp(g)
        swait_grp(NGRP - 1)

    return k(idx, table)


def _dense_block(src_ref, tgt_ref, w2e_ref, v2_ref, w2d_ref, k2_ref,
                 outa_ref, outb_ref, *, bb):
    sh = bb * S // 2
    x = src_ref[...]
    h2 = jnp.tanh(jnp.dot(x, w2e_ref[...],
                          preferred_element_type=jnp.float32))
    lg = jnp.dot(h2, v2_ref[...],
                 preferred_element_type=jnp.float32)
    l3 = lg.reshape(bb, S // 2, 2)
    m = jnp.max(jnp.max(l3, axis=1, keepdims=True),
                axis=2, keepdims=True)
    e = jnp.exp(l3 - m)
    ssum = jnp.sum(jnp.sum(e, axis=1, keepdims=True),
                   axis=2, keepdims=True)
    attn2 = (e / ssum).reshape(sh, 2)
    arep = jnp.dot(attn2, k2_ref[...],
                   preferred_element_type=jnp.float32)
    c128 = jnp.sum((h2 * arep).reshape(bb, S // 2, 2 * D), axis=1)
    ctx = c128[:, :D] + c128[:, D:]

    t2 = tgt_ref[...]
    th2 = jnp.tanh(jnp.dot(t2, w2d_ref[...],
                           preferred_element_type=jnp.float32))
    cc = jnp.concatenate([ctx, ctx], axis=1)
    prod = (th2.reshape(bb, T // 2, 2 * D)
            * jnp.broadcast_to(cc.reshape(bb, 1, 2 * D),
                               (bb, T // 2, 2 * D)))
    outa_ref[...] = jnp.sum(prod[:, :, :D], axis=2)
    outb_ref[...] = jnp.sum(prod[:, :, D:], axis=2)


def _dense_tc(emb2, W_enc, v_attn, W_dec):
    bb = 64
    grid = B // bb
    sh = bb * S // 2
    th = bb * T // 2
    tgt_block0 = (B * S // 2) // th

    z = jnp.zeros((D, D), jnp.float32)
    w2e = jnp.block([[W_enc, z], [z, W_enc]])
    w2d = jnp.block([[W_dec, z], [z, W_dec]])
    zv = jnp.zeros((D, 1), jnp.float32)
    v2 = jnp.block([[v_attn, zv], [zv, v_attn]])
    o = jnp.ones((1, D), jnp.float32)
    zo = jnp.zeros((1, D), jnp.float32)
    k2 = jnp.block([[o, zo], [zo, o]])

    outa, outb = pl.pallas_call(
        functools.partial(_dense_block, bb=bb),
        grid=(grid,),
        in_specs=[
            pl.BlockSpec((sh, 2 * D), lambda i: (i, 0)),
            pl.BlockSpec((th, 2 * D), lambda i: (tgt_block0 + i, 0)),
            pl.BlockSpec((2 * D, 2 * D), lambda i: (0, 0)),
            pl.BlockSpec((2 * D, 2), lambda i: (0, 0)),
            pl.BlockSpec((2 * D, 2 * D), lambda i: (0, 0)),
            pl.BlockSpec((2, 2 * D), lambda i: (0, 0)),
        ],
        out_specs=[
            pl.BlockSpec((bb, T // 2), lambda i: (i, 0)),
            pl.BlockSpec((bb, T // 2), lambda i: (i, 0)),
        ],
        out_shape=[
            jax.ShapeDtypeStruct((B, T // 2), jnp.float32),
            jax.ShapeDtypeStruct((B, T // 2), jnp.float32),
        ],
    )(emb2, emb2, w2e, v2, w2d, k2)
    return jnp.stack([outa, outb], axis=2).reshape(B, T)


def kernel(source, target, word_lut, W_enc, v_attn, W_dec):
    wl2 = _transpose_tc(word_lut.T)
    wl_lin = wl2.reshape(V_LIN, D)
    padv = jnp.arange(N_PAD - N_ROWS, dtype=jnp.int32)
    idx = jnp.concatenate([source.reshape(-1), target.reshape(-1), padv])
    tb = idx & (VB - 1)
    idx = ((idx & ~(VB - 1)) | (((tb << 1) & (VB - 1)) | (tb >> 13)))
    idx = idx.reshape(NW, CPW, CHUNK)
    emb = _gather_sc(idx, wl_lin)
    emb2 = emb.reshape(N_PAD // 2, 2 * D)
    return _dense_tc(emb2, W_enc, v_attn.reshape(D, 1), W_dec)


# speedup vs baseline: 2.5344x; 1.0343x over previous
import functools

import jax
import jax.numpy as jnp
from jax import lax
from jax.experimental import pallas as pl
from jax.experimental.pallas import tpu as pltpu
from jax.experimental.pallas import tpu_sc as plsc

VOCAB = 1000000
D = 64
B = 1024
S = 200
T = 50

NC = 2
NS = 16
NW = NC * NS

BH = B // 2
CHUNK = 128
NR_H = BH * (S + T)
CPW = -(-NR_H // (NW * CHUNK))
N_PAD = NW * CPW * CHUNK
G = 4
NGRP = CPW // G
NBUFG = 3

VB = 16384
HB = VB // 2
NVB = -(-VOCAB // VB)
V_LIN = NVB * VB


def _transpose_block(in_ref, out_ref):
    x = in_ref[...]
    xt = x.T
    out_ref[...] = jnp.concatenate([xt[:HB], xt[HB:]], axis=1)


def _transpose_tc(wl_t):
    return pl.pallas_call(
        _transpose_block,
        grid=(NVB,),
        in_specs=[pl.BlockSpec((D, VB), lambda i: (0, i))],
        out_specs=pl.BlockSpec((HB, 2 * D), lambda i: (i, 0)),
        out_shape=jax.ShapeDtypeStruct((V_LIN // 2, 2 * D), jnp.float32),
    )(wl_t)


def _gather_sc(idx, table):
    mesh = plsc.VectorSubcoreMesh(core_axis_name="c", subcore_axis_name="s")
    NFLY = NBUFG - 1

    @functools.partial(
        pl.kernel,
        mesh=mesh,
        out_type=jax.ShapeDtypeStruct((NW * CPW, CHUNK, D), jnp.float32),
        scratch_types=[
            pltpu.VMEM((CPW, CHUNK), jnp.int32),
            pltpu.VMEM((NBUFG * G, CHUNK, D), jnp.float32),
            pltpu.SemaphoreType.DMA((NBUFG,)),
            pltpu.SemaphoreType.DMA((NBUFG,)),
        ],
        compiler_params=pltpu.CompilerParams(use_tc_tiling_on_sc=False),
    )
    def k(idx_hbm, table_hbm, out_hbm, idx_v, bufs, sem_g, sem_s):
        wid = lax.axis_index("s") * NC + lax.axis_index("c")
        pltpu.sync_copy(idx_hbm.at[wid], idx_v)
        base = wid * CPW

        def gather_grp(g):
            k = lax.rem(g, NBUFG)
            for i in range(G):
                pltpu.async_copy(
                    table_hbm.at[idx_v.at[g * G + i]], bufs.at[k * G + i],
                    sem_g.at[k])

        def gwait_grp(g):
            k = lax.rem(g, NBUFG)
            for _ in range(G):
                pltpu.make_async_copy(
                    table_hbm.at[pl.ds(0, CHUNK)], bufs.at[0],
                    sem_g.at[k]).wait()

        def store_grp(g):
            k = lax.rem(g, NBUFG)
            for i in range(G):
                pltpu.async_copy(
                    bufs.at[k * G + i], out_hbm.at[base + g * G + i],
                    sem_s.at[k])

        def swait_grp(g):
            k = lax.rem(g, NBUFG)
            for _ in range(G):
                pltpu.make_async_copy(
                    table_hbm.at[pl.ds(0, CHUNK)], bufs.at[0],
                    sem_s.at[k]).wait()

        for g in range(NFLY):
            gather_grp(g)
        gwait_grp(0)
        store_grp(0)
        gather_grp(NFLY)

        def body(g, carry):
            gwait_grp(g)
            swait_grp(g - 1)
            store_grp(g)
            gather_grp(g + NFLY)
            return carry

        lax.fori_loop(1, NGRP - NFLY, body, 0)

        for g in range(NGRP - NFLY, NGRP):
            gwait_grp(g)
            swait_grp(g - 1)
            store_grp(g)
        swait_grp(NGRP - 1)

    return k(idx, table)


def _dense_block(src_ref, tgt_ref, w2e_ref, v2_ref, w2d_ref, k2_ref,
                 outa_ref, outb_ref, *, bb):
    sh = bb * S // 2
    x = src_ref[...]
    h2 = jnp.tanh(jnp.dot(x, w2e_ref[...],
                          preferred_element_type=jnp.float32))
    lg = jnp.dot(h2, v2_ref[...],
                 preferred_element_type=jnp.float32)
    l3 = lg.reshape(bb, S // 2, 2)
    m = jnp.max(jnp.max(l3, axis=1, keepdims=True),
                axis=2, keepdims=True)
    e = jnp.exp(l3 - m)
    ssum = jnp.sum(jnp.sum(e, axis=1, keepdims=True),
                   axis=2, keepdims=True)
    attn2 = (e / ssum).reshape(sh, 2)
    arep = jnp.dot(attn2, k2_ref[...],
                   preferred_element_type=jnp.float32)
    c128 = jnp.sum((h2 * arep).reshape(bb, S // 2, 2 * D), axis=1)
    ctx = c128[:, :D] + c128[:, D:]

    t2 = tgt_ref[...]
    th2 = jnp.tanh(jnp.dot(t2, w2d_ref[...],
                           preferred_element_type=jnp.float32))
    cc = jnp.concatenate([ctx, ctx], axis=1)
    prod = (th2.reshape(bb, T // 2, 2 * D)
            * jnp.broadcast_to(cc.reshape(bb, 1, 2 * D),
                               (bb, T // 2, 2 * D)))
    outa_ref[...] = jnp.sum(prod[:, :, :D], axis=2)
    outb_ref[...] = jnp.sum(prod[:, :, D:], axis=2)


def _dense_tc(emb2, weights):
    w2e, v2, w2d, k2 = weights
    bb = 64
    grid = BH // bb
    sh = bb * S // 2
    th = bb * T // 2
    tgt_block0 = (BH * S // 2) // th
    outa, outb = pl.pallas_call(
        functools.partial(_dense_block, bb=bb),
        grid=(grid,),
        in_specs=[
            pl.BlockSpec((sh, 2 * D), lambda i: (i, 0)),
            pl.BlockSpec((th, 2 * D), lambda i: (tgt_block0 + i, 0)),
            pl.BlockSpec((2 * D, 2 * D), lambda i: (0, 0)),
            pl.BlockSpec((2 * D, 2), lambda i: (0, 0)),
            pl.BlockSpec((2 * D, 2 * D), lambda i: (0, 0)),
            pl.BlockSpec((2, 2 * D), lambda i: (0, 0)),
        ],
        out_specs=[
            pl.BlockSpec((bb, T // 2), lambda i: (i, 0)),
            pl.BlockSpec((bb, T // 2), lambda i: (i, 0)),
        ],
        out_shape=[
            jax.ShapeDtypeStruct((BH, T // 2), jnp.float32),
            jax.ShapeDtypeStruct((BH, T // 2), jnp.float32),
        ],
    )(emb2, emb2, w2e, v2, w2d, k2)
    return jnp.stack([outa, outb], axis=2).reshape(BH, T)


def _half_idx(src_h, tgt_h, pad_base):
    padv = pad_base + jnp.arange(N_PAD - NR_H, dtype=jnp.int32)
    idx = jnp.concatenate([src_h.reshape(-1), tgt_h.reshape(-1), padv])
    tb = idx & (VB - 1)
    idx = ((idx & ~(VB - 1)) | (((tb << 1) & (VB - 1)) | (tb >> 13)))
    return idx.reshape(NW, CPW, CHUNK)


def kernel(source, target, word_lut, W_enc, v_attn, W_dec):
    wl2 = _transpose_tc(word_lut.T)
    wl_lin = wl2.reshape(V_LIN, D)

    idx_a = _half_idx(source[:BH], target[:BH], 0)
    idx_b = _half_idx(source[BH:], target[BH:], 8192)
    emb_a = _gather_sc(idx_a, wl_lin)
    emb_b = _gather_sc(idx_b, wl_lin)

    va = v_attn.reshape(D, 1)
    z = jnp.zeros((D, D), jnp.float32)
    w2e = jnp.block([[W_enc, z], [z, W_enc]])
    w2d = jnp.block([[W_dec, z], [z, W_dec]])
    zv = jnp.zeros((D, 1), jnp.float32)
    v2 = jnp.block([[va, zv], [zv, va]])
    o = jnp.ones((1, D), jnp.float32)
    zo = jnp.zeros((1, D), jnp.float32)
    k2 = jnp.block([[o, zo], [zo, o]])
    weights = (w2e, v2, w2d, k2)

    out_a = _dense_tc(emb_a.reshape(N_PAD // 2, 2 * D), weights)
    out_b = _dense_tc(emb_b.reshape(N_PAD // 2, 2 * D), weights)
    return jnp.concatenate([out_a, out_b], axis=0)


# speedup vs baseline: 2.6384x; 1.0410x over previous
import functools

import jax
import jax.numpy as jnp
from jax import lax
from jax.experimental import pallas as pl
from jax.experimental.pallas import tpu as pltpu
from jax.experimental.pallas import tpu_sc as plsc

VOCAB = 1000000
D = 64
B = 1024
S = 200
T = 50

NC = 2
NS = 16
NW = NC * NS

BH = B // 2
CHUNK = 128
NR_H = BH * (S + T)
CPW = -(-NR_H // (NW * CHUNK))
N_PAD = NW * CPW * CHUNK
G = 4
NGRP = CPW // G
NBUFG = 3

VB = 32768
HB = VB // 2
NVB = -(-VOCAB // VB)
V_LIN = NVB * VB


def _transpose_block(in_ref, out_ref):
    x = in_ref[...]
    xt = x.T
    out_ref[...] = jnp.concatenate([xt[:HB], xt[HB:]], axis=1)


def _transpose_tc(wl_t):
    return pl.pallas_call(
        _transpose_block,
        grid=(NVB,),
        in_specs=[pl.BlockSpec((D, VB), lambda i: (0, i))],
        out_specs=pl.BlockSpec((HB, 2 * D), lambda i: (i, 0)),
        out_shape=jax.ShapeDtypeStruct((V_LIN // 2, 2 * D), jnp.float32),
    )(wl_t)


def _gather_sc(idx, table):
    mesh = plsc.VectorSubcoreMesh(core_axis_name="c", subcore_axis_name="s")
    NFLY = NBUFG - 1

    @functools.partial(
        pl.kernel,
        mesh=mesh,
        out_type=jax.ShapeDtypeStruct((NW * CPW, CHUNK, D), jnp.float32),
        scratch_types=[
            pltpu.VMEM((CPW, CHUNK), jnp.int32),
            pltpu.VMEM((NBUFG * G, CHUNK, D), jnp.float32),
            pltpu.SemaphoreType.DMA((NBUFG,)),
            pltpu.SemaphoreType.DMA((NBUFG,)),
        ],
        compiler_params=pltpu.CompilerParams(use_tc_tiling_on_sc=False),
    )
    def k(idx_hbm, table_hbm, out_hbm, idx_v, bufs, sem_g, sem_s):
        wid = lax.axis_index("s") * NC + lax.axis_index("c")
        pltpu.sync_copy(idx_hbm.at[wid], idx_v)
        base = wid * CPW

        def gather_grp(g):
            k = lax.rem(g, NBUFG)
            for i in range(G):
                pltpu.async_copy(
                    table_hbm.at[idx_v.at[g * G + i]], bufs.at[k * G + i],
                    sem_g.at[k])

        def gwait_grp(g):
            k = lax.rem(g, NBUFG)
            for _ in range(G):
                pltpu.make_async_copy(
                    table_hbm.at[pl.ds(0, CHUNK)], bufs.at[0],
                    sem_g.at[k]).wait()

        def store_grp(g):
            k = lax.rem(g, NBUFG)
            for i in range(G):
                pltpu.async_copy(
                    bufs.at[k * G + i], out_hbm.at[base + g * G + i],
                    sem_s.at[k])

        def swait_grp(g):
            k = lax.rem(g, NBUFG)
            for _ in range(G):
                pltpu.make_async_copy(
                    table_hbm.at[pl.ds(0, CHUNK)], bufs.at[0],
                    sem_s.at[k]).wait()

        for g in range(NFLY):
            gather_grp(g)
        gwait_grp(0)
        store_grp(0)
        gather_grp(NFLY)

        def body(g, carry):
            gwait_grp(g)
            swait_grp(g - 1)
            store_grp(g)
            gather_grp(g + NFLY)
            return carry

        lax.fori_loop(1, NGRP - NFLY, body, 0)

        for g in range(NGRP - NFLY, NGRP):
            gwait_grp(g)
            swait_grp(g - 1)
            store_grp(g)
        swait_grp(NGRP - 1)

    return k(idx, table)


def _dense_block(src_ref, tgt_ref, w2e_ref, v2_ref, w2d_ref, k2_ref,
                 outa_ref, outb_ref, *, bb):
    sh = bb * S // 2
    x = src_ref[...]
    h2 = jnp.tanh(jnp.dot(x, w2e_ref[...],
                          preferred_element_type=jnp.float32))
    lg = jnp.dot(h2, v2_ref[...],
                 preferred_element_type=jnp.float32)
    l3 = lg.reshape(bb, S // 2, 2)
    m = jnp.max(jnp.max(l3, axis=1, keepdims=True),
                axis=2, keepdims=True)
    e = jnp.exp(l3 - m)
    ssum = jnp.sum(jnp.sum(e, axis=1, keepdims=True),
                   axis=2, keepdims=True)
    attn2 = (e / ssum).reshape(sh, 2)
    arep = jnp.dot(attn2, k2_ref[...],
                   preferred_element_type=jnp.float32)
    c128 = jnp.sum((h2 * arep).reshape(bb, S // 2, 2 * D), axis=1)
    ctx = c128[:, :D] + c128[:, D:]

    t2 = tgt_ref[...]
    th2 = jnp.tanh(jnp.dot(t2, w2d_ref[...],
                           preferred_element_type=jnp.float32))
    cc = jnp.concatenate([ctx, ctx], axis=1)
    prod = (th2.reshape(bb, T // 2, 2 * D)
            * jnp.broadcast_to(cc.reshape(bb, 1, 2 * D),
                               (bb, T // 2, 2 * D)))
    outa_ref[...] = jnp.sum(prod[:, :, :D], axis=2)
    outb_ref[...] = jnp.sum(prod[:, :, D:], axis=2)


def _dense_tc(emb2, weights):
    w2e, v2, w2d, k2 = weights
    bb = 64
    grid = BH // bb
    sh = bb * S // 2
    th = bb * T // 2
    tgt_block0 = (BH * S // 2) // th
    outa, outb = pl.pallas_call(
        functools.partial(_dense_block, bb=bb),
        grid=(grid,),
        in_specs=[
            pl.BlockSpec((sh, 2 * D), lambda i: (i, 0)),
            pl.BlockSpec((th, 2 * D), lambda i: (tgt_block0 + i, 0)),
            pl.BlockSpec((2 * D, 2 * D), lambda i: (0, 0)),
            pl.BlockSpec((2 * D, 2), lambda i: (0, 0)),
            pl.BlockSpec((2 * D, 2 * D), lambda i: (0, 0)),
            pl.BlockSpec((2, 2 * D), lambda i: (0, 0)),
        ],
        out_specs=[
            pl.BlockSpec((bb, T // 2), lambda i: (i, 0)),
            pl.BlockSpec((bb, T // 2), lambda i: (i, 0)),
        ],
        out_shape=[
            jax.ShapeDtypeStruct((BH, T // 2), jnp.float32),
            jax.ShapeDtypeStruct((BH, T // 2), jnp.float32),
        ],
    )(emb2, emb2, w2e, v2, w2d, k2)
    return jnp.stack([outa, outb], axis=2).reshape(BH, T)


def _half_idx(src_h, tgt_h, pad_base):
    padv = pad_base + jnp.arange(N_PAD - NR_H, dtype=jnp.int32)
    idx = jnp.concatenate([src_h.reshape(-1), tgt_h.reshape(-1), padv])
    tb = idx & (VB - 1)
    idx = ((idx & ~(VB - 1))
           | (((tb << 1) & (VB - 1)) | (tb >> (VB.bit_length() - 2))))
    return idx.reshape(NW, CPW, CHUNK)


def kernel(source, target, word_lut, W_enc, v_attn, W_dec):
    wl2 = _transpose_tc(word_lut.T)
    wl_lin = wl2.reshape(V_LIN, D)

    idx_a = _half_idx(source[:BH], target[:BH], 0)
    idx_b = _half_idx(source[BH:], target[BH:], 8192)
    emb_a = _gather_sc(idx_a, wl_lin)
    emb_b = _gather_sc(idx_b, wl_lin)

    va = v_attn.reshape(D, 1)
    z = jnp.zeros((D, D), jnp.float32)
    w2e = jnp.block([[W_enc, z], [z, W_enc]])
    w2d = jnp.block([[W_dec, z], [z, W_dec]])
    zv = jnp.zeros((D, 1), jnp.float32)
    v2 = jnp.block([[va, zv], [zv, va]])
    o = jnp.ones((1, D), jnp.float32)
    zo = jnp.zeros((1, D), jnp.float32)
    k2 = jnp.block([[o, zo], [zo, o]])
    weights = (w2e, v2, w2d, k2)

    out_a = _dense_tc(emb_a.reshape(N_PAD // 2, 2 * D), weights)
    out_b = _dense_tc(emb_b.reshape(N_PAD // 2, 2 * D), weights)
    return jnp.concatenate([out_a, out_b], axis=0)


# speedup vs baseline: 2.6740x; 1.0135x over previous
import functools

import jax
import jax.numpy as jnp
from jax import lax
from jax.experimental import pallas as pl
from jax.experimental.pallas import tpu as pltpu
from jax.experimental.pallas import tpu_sc as plsc

VOCAB = 1000000
D = 64
B = 1024
S = 200
T = 50

NC = 2
NS = 16
NW = NC * NS

BH = B // 2
CHUNK = 128
NR_H = BH * (S + T)
CPW = -(-NR_H // (NW * CHUNK))
N_PAD = NW * CPW * CHUNK
G = 4
NGRP = CPW // G
NBUFG = 3

VB = 32768
HB = VB // 2
NVB = -(-VOCAB // VB)
V_LIN = NVB * VB


def _transpose_block(in_ref, out_ref):
    x = in_ref[...]
    xt = x.T
    out_ref[...] = jnp.concatenate([xt[:HB], xt[HB:]], axis=1)


def _transpose_tc(wl_t):
    return pl.pallas_call(
        _transpose_block,
        grid=(NVB,),
        in_specs=[pl.BlockSpec((D, VB), lambda i: (0, i))],
        out_specs=pl.BlockSpec((HB, 2 * D), lambda i: (i, 0)),
        out_shape=jax.ShapeDtypeStruct((V_LIN // 2, 2 * D), jnp.float32),
    )(wl_t)


def _gather_sc(idx, table):
    mesh = plsc.VectorSubcoreMesh(core_axis_name="c", subcore_axis_name="s")
    NFLY = NBUFG - 1

    @functools.partial(
        pl.kernel,
        mesh=mesh,
        out_type=jax.ShapeDtypeStruct((NW * CPW, CHUNK, D), jnp.float32),
        scratch_types=[
            pltpu.VMEM((CPW, CHUNK), jnp.int32),
            pltpu.VMEM((NBUFG * G, CHUNK, D), jnp.float32),
            pltpu.SemaphoreType.DMA((NBUFG,)),
            pltpu.SemaphoreType.DMA((NBUFG,)),
        ],
        compiler_params=pltpu.CompilerParams(use_tc_tiling_on_sc=False),
    )
    def k(idx_hbm, table_hbm, out_hbm, idx_v, bufs, sem_g, sem_s):
        wid = lax.axis_index("s") * NC + lax.axis_index("c")
        pltpu.sync_copy(idx_hbm.at[wid], idx_v)
        base = wid * CPW

        def gather_grp(g):
            k = lax.rem(g, NBUFG)
            for i in range(G):
                pltpu.async_copy(
                    table_hbm.at[idx_v.at[g * G + i]], bufs.at[k * G + i],
                    sem_g.at[k])

        def gwait_grp(g):
            k = lax.rem(g, NBUFG)
            for _ in range(G):
                pltpu.make_async_copy(
                    table_hbm.at[pl.ds(0, CHUNK)], bufs.at[0],
                    sem_g.at[k]).wait()

        def store_grp(g):
            k = lax.rem(g, NBUFG)
            for i in range(G):
                pltpu.async_copy(
                    bufs.at[k * G + i], out_hbm.at[base + g * G + i],
                    sem_s.at[k])

        def swait_grp(g):
            k = lax.rem(g, NBUFG)
            for _ in range(G):
                pltpu.make_async_copy(
                    table_hbm.at[pl.ds(0, CHUNK)], bufs.at[0],
                    sem_s.at[k]).wait()

        for g in range(NFLY):
            gather_grp(g)
        gwait_grp(0)
        store_grp(0)
        gather_grp(NFLY)

        def body(g, carry):
            gwait_grp(g)
            swait_grp(g - 1)
            store_grp(g)
            gather_grp(g + NFLY)
            return carry

        lax.fori_loop(1, NGRP - NFLY, body, 0)

        for g in range(NGRP - NFLY, NGRP):
            gwait_grp(g)
            swait_grp(g - 1)
            store_grp(g)
        swait_grp(NGRP - 1)

    return k(idx, table)


def _dense_block(src_ref, tgt_ref, w2e_ref, v2_ref, w2d_ref, k2_ref,
                 outa_ref, outb_ref, *, bb):
    sh = bb * S // 2
    x = src_ref[...]
    h2 = jnp.tanh(jnp.dot(x, w2e_ref[...],
                          preferred_element_type=jnp.float32))
    lg = jnp.dot(h2, v2_ref[...],
                 preferred_element_type=jnp.float32)
    l3 = lg.reshape(bb, S // 2, 2)
    e = jnp.exp(l3)
    ssum = jnp.sum(jnp.sum(e, axis=1, keepdims=True),
                   axis=2, keepdims=True)
    attn2 = (e / ssum).reshape(sh, 2)
    arep = jnp.dot(attn2, k2_ref[...],
                   preferred_element_type=jnp.float32)
    c128 = jnp.sum((h2 * arep).reshape(bb, S // 2, 2 * D), axis=1)
    ctx = c128[:, :D] + c128[:, D:]

    t2 = tgt_ref[...]
    th2 = jnp.tanh(jnp.dot(t2, w2d_ref[...],
                           preferred_element_type=jnp.float32))
    cc = jnp.concatenate([ctx, ctx], axis=1)
    prod = (th2.reshape(bb, T // 2, 2 * D)
            * jnp.broadcast_to(cc.reshape(bb, 1, 2 * D),
                               (bb, T // 2, 2 * D)))
    outa_ref[...] = jnp.sum(prod[:, :, :D], axis=2)
    outb_ref[...] = jnp.sum(prod[:, :, D:], axis=2)


def _dense_tc(emb2, weights):
    w2e, v2, w2d, k2 = weights
    bb = 64
    grid = BH // bb
    sh = bb * S // 2
    th = bb * T // 2
    tgt_block0 = (BH * S // 2) // th
    outa, outb = pl.pallas_call(
        functools.partial(_dense_block, bb=bb),
        grid=(grid,),
        in_specs=[
            pl.BlockSpec((sh, 2 * D), lambda i: (i, 0)),
            pl.BlockSpec((th, 2 * D), lambda i: (tgt_block0 + i, 0)),
            pl.BlockSpec((2 * D, 2 * D), lambda i: (0, 0)),
            pl.BlockSpec((2 * D, 2), lambda i: (0, 0)),
            pl.BlockSpec((2 * D, 2 * D), lambda i: (0, 0)),
            pl.BlockSpec((2, 2 * D), lambda i: (0, 0)),
        ],
        out_specs=[
            pl.BlockSpec((bb, T // 2), lambda i: (i, 0)),
            pl.BlockSpec((bb, T // 2), lambda i: (i, 0)),
        ],
        out_shape=[
            jax.ShapeDtypeStruct((BH, T // 2), jnp.float32),
            jax.ShapeDtypeStruct((BH, T // 2), jnp.float32),
        ],
    )(emb2, emb2, w2e, v2, w2d, k2)
    return jnp.stack([outa, outb], axis=2).reshape(BH, T)


def _half_idx(src_h, tgt_h, pad_base):
    padv = pad_base + jnp.arange(N_PAD - NR_H, dtype=jnp.int32)
    idx = jnp.concatenate([src_h.reshape(-1), tgt_h.reshape(-1), padv])
    tb = idx & (VB - 1)
    idx = ((idx & ~(VB - 1))
           | (((tb << 1) & (VB - 1)) | (tb >> (VB.bit_length() - 2))))
    return idx.reshape(NW, CPW, CHUNK)


def kernel(source, target, word_lut, W_enc, v_attn, W_dec):
    wl2 = _transpose_tc(word_lut.T)
    wl_lin = wl2.reshape(V_LIN, D)

    idx_a = _half_idx(source[:BH], target[:BH], 0)
    idx_b = _half_idx(source[BH:], target[BH:], 8192)
    emb_a = _gather_sc(idx_a, wl_lin)
    emb_b = _gather_sc(idx_b, wl_lin)

    va = v_attn.reshape(D, 1)
    z = jnp.zeros((D, D), jnp.float32)
    w2e = jnp.block([[W_enc, z], [z, W_enc]])
    w2d = jnp.block([[W_dec, z], [z, W_dec]])
    zv = jnp.zeros((D, 1), jnp.float32)
    v2 = jnp.block([[va, zv], [zv, va]])
    o = jnp.ones((1, D), jnp.float32)
    zo = jnp.zeros((1, D), jnp.float32)
    k2 = jnp.block([[o, zo], [zo, o]])
    weights = (w2e, v2, w2d, k2)

    out_a = _dense_tc(emb_a.reshape(N_PAD // 2, 2 * D), weights)
    out_b = _dense_tc(emb_b.reshape(N_PAD // 2, 2 * D), weights)
    return jnp.concatenate([out_a, out_b], axis=0)
